# baseline (speedup 1.0000x reference)
import functools

import jax
import jax.numpy as jnp
from jax import lax
from jax.experimental import pallas as pl
from jax.experimental.pallas import tpu as pltpu
from jax.experimental.pallas import tpu_sc as plsc

_NC = 2
_NS = 16
_NW = _NC * _NS
_NBUF = 2


@functools.partial(jax.jit, static_argnames=("chunk",))
def _sc_gather(x, idx, *, chunk):
    b = idx.shape[0]
    d = x.shape[1]
    units, rem = divmod(b // 8, _NW)
    n_long = (b - 8 * units * _NW) // 8
    span_l = 8 * (units + 1)
    span_s = 8 * units
    assert b % 8 == 0 and rem == n_long and 0 < n_long <= _NW

    sizes = [chunk] * (span_l // chunk)
    if span_l % chunk:
        sizes.append(span_l % chunk)
    offs = [sum(sizes[:g]) for g in range(len(sizes))]
    n_chunks = len(sizes)
    assert sizes[-1] > 8 and all(s % 8 == 0 for s in sizes)

    mesh = plsc.VectorSubcoreMesh(core_axis_name="c", subcore_axis_name="s")

    @functools.partial(
        pl.kernel,
        mesh=mesh,
        out_type=jax.ShapeDtypeStruct((b, d), jnp.float32),
        scratch_types=[
            pltpu.VMEM((span_l,), jnp.int32),
            pltpu.VMEM((_NBUF, chunk, d), jnp.float32),
        ]
        + [pltpu.SemaphoreType.DMA] * _NBUF,
    )
    def k(table_hbm, idx_hbm, out_hbm, idx_v, bufs, *sems):
        wid = lax.axis_index("s") * _NC + lax.axis_index("c")
        base = span_s * wid + 8 * jnp.minimum(wid, n_long)
        is_long = wid < n_long

        @pl.when(jnp.logical_not(is_long))
        def _():
            idx_v[pl.ds(span_l - 16, 16)] = jnp.zeros((16,), jnp.int32)
            pltpu.sync_copy(idx_hbm.at[pl.ds(base, span_s)],
                            idx_v.at[pl.ds(0, span_s)])

        @pl.when(is_long)
        def _():
            pltpu.sync_copy(idx_hbm.at[pl.ds(base, span_l)], idx_v)

        def start_gather(g):
            return pltpu.async_copy(
                table_hbm.at[idx_v.at[pl.ds(offs[g], sizes[g])]],
                bufs.at[g % _NBUF].at[pl.ds(0, sizes[g])],
                sems[g % _NBUF],
            )

        copies = [None] * _NBUF
        for g in range(min(_NBUF, n_chunks)):
            copies[g] = start_gather(g)
        for g in range(n_chunks):
            cur = g % _NBUF
            copies[cur].wait()
            if g + 1 < n_chunks:
                pltpu.sync_copy(
                    bufs.at[cur].at[pl.ds(0, sizes[g])],
                    out_hbm.at[pl.ds(base + offs[g], sizes[g])],
                )
            else:

                @pl.when(is_long)
                def _():
                    pltpu.sync_copy(
                        bufs.at[cur].at[pl.ds(0, sizes[g])],
                        out_hbm.at[pl.ds(base + offs[g], sizes[g])],
                    )

                @pl.when(jnp.logical_not(is_long))
                def _():
                    pltpu.sync_copy(
                        bufs.at[cur].at[pl.ds(0, sizes[g] - 8)],
                        out_hbm.at[pl.ds(base + offs[g], sizes[g] - 8)],
                    )
            if g + _NBUF < n_chunks:
                copies[cur] = start_gather(g + _NBUF)

    return k(x, idx)


def kernel(x, coarse_idx):
    return _sc_gather(x, coarse_idx.astype(jnp.int32), chunk=240)
